# baseline (speedup 1.0000x reference)
import functools

import jax
import jax.numpy as jnp
from jax import lax
from jax.experimental import pallas as pl
from jax.experimental.pallas import tpu as pltpu
from jax.experimental.pallas import tpu_sc as plsc

D = 128
NC, NS, L = 2, 16, 16
G = 128
F32 = jnp.float32
I32 = jnp.int32


@functools.cache
def _mesh():
    return plsc.VectorSubcoreMesh(core_axis_name="c", subcore_axis_name="s",
                                  num_cores=NC, num_subcores=NS)


def _ru(x, m):
    return (x + m - 1) // m * m


def _static_spans(total, step):
    out = []
    off = 0
    while off < total:
        w = min(step, total - off)
        out.append((off, w))
        off += w
    return out


def _make_counts(ep, sizes):
    rpe = ep // 32 // NS
    amax = max(sizes)

    def body(d0, d1, d2, d3, o0, o1, o2, o3, acc, dv, ones, zb, cb, sem):
        del sem
        cid = lax.axis_index("c")
        sid = lax.axis_index("s")
        def init16(i, c):
            zb[0, pl.ds(i * L, L)] = jnp.zeros((L,), F32)
            ones[0, pl.ds(lax.rem(i, jnp.int32(2)) * L, L)] = (
                jnp.ones((L,), F32))
            return c

        lax.fori_loop(0, 1024 // L, init16, 0)

        def job(dst_hbm, out_hbm, n):
            span = n // NS
            base = sid * span
            for (off, w) in _static_spans(span, 1024):
                pltpu.sync_copy(zb.at[0, pl.ds(0, w)],
                                acc.at[pl.ds(base + off, w)])
            plsc.subcore_barrier()
            pltpu.sync_copy(dst_hbm.at[pl.ds(sid * rpe, rpe)], dv)

            def it(j, c):
                pltpu.sync_copy(ones.at[0], acc.at[dv.at[j]], add=True)
                return c

            lax.fori_loop(0, rpe, it, 0)
            plsc.subcore_barrier()
            pltpu.sync_copy(acc.at[pl.ds(base, span)], cb.at[pl.ds(0, span)])
            pltpu.sync_copy(cb.at[pl.ds(0, span)],
                            out_hbm.at[pl.ds(base, span)])
            plsc.subcore_barrier()

        @pl.when(cid == 0)
        def _():
            job(d0, o0, sizes[0])
            job(d1, o1, sizes[1])

        @pl.when(cid == 1)
        def _():
            job(d2, o2, sizes[2])
            job(d3, o3, sizes[3])

    return pl.kernel(
        body,
        out_type=[jax.ShapeDtypeStruct((s,), F32) for s in sizes],
        mesh=_mesh(),
        scratch_types=[
            pltpu.VMEM_SHARED((amax,), F32),
            pltpu.VMEM((rpe, 32), I32),
            pltpu.VMEM((1, 32), F32),
            pltpu.VMEM((1, 1024), F32),
            pltpu.VMEM((amax // NS,), F32),
            pltpu.SemaphoreType.DMA,
        ],
    )


def _make_segsum(ep, jobs):
    gw = 32
    eb = 32
    upt = ep // gw // NS
    nblk = upt // eb
    amax = max(nd_p // (NC * npc) for (nd_p, npc) in jobs) + L
    assert upt % eb == 0

    def body(*refs):
        nj = len(jobs)
        xs = refs[0:nj]
        srcs = refs[nj:2 * nj]
        dsts = refs[2 * nj:3 * nj]
        outs = refs[3 * nj:4 * nj]
        (acc, sv, dv, r0, r1, r2, x0, x1, x2, zb,
         g0, g1, g2, s0, s1, s2, zs) = refs[4 * nj:]
        rows = (r0, r1, r2)
        sidx = (x0, x1, x2)
        gs = (g0, g1, g2)
        ss = (s0, s1, s2)
        cid = lax.axis_index("c")
        sid = lax.axis_index("s")

        def zinit(i, c):
            zb[i // (D // L), pl.ds(lax.rem(i, jnp.int32(D // L)) * L, L)] = (
                jnp.zeros((L,), F32))
            return c

        lax.fori_loop(0, 8 * (D // L), zinit, 0)
        cp = pltpu.async_copy

        for j, (nd_p, npc) in enumerate(jobs):
            x_hbm, src_hbm, dst_hbm, out_hbm = xs[j], srcs[j], dsts[j], outs[j]
            chunk = nd_p // (NC * npc)
            rpt = chunk // NS
            nz = rpt // 8
            for cj in range(npc):
                lo = (cid * npc + cj) * chunk
                def ziss(i, c):
                    cp(zb, acc.at[pl.ds(sid * rpt + i * 8, 8)], zs)
                    return c

                def zdrn(i, c):
                    pltpu.make_async_copy(
                        zb, acc.at[pl.ds(sid * rpt, 8)], zs).wait()
                    return c

                lax.fori_loop(0, nz, ziss, 0)
                lax.fori_loop(0, nz, zdrn, 0)
                plsc.subcore_barrier()

                def route(u, b):
                    for k in range(gw // L):
                        dd = dv[u, pl.ds(k * L, L)]
                        ok = (dd >= lo) & (dd < lo + chunk)
                        sidx[b][pl.ds(k * L, L)] = jnp.where(
                            ok, dd - lo, jnp.int32(chunk))

                def unit(u, b, wait_prev, issue_next):
                    pltpu.make_async_copy(x_hbm.at[sv.at[0]], rows[b],
                                          gs[b]).wait()
                    route(u, b)
                    cp(rows[b], acc.at[sidx[b]], ss[b], add=True)
                    bp = (b + 2) % 3
                    if wait_prev:
                        pltpu.make_async_copy(rows[bp], acc.at[sidx[bp]],
                                              ss[bp]).wait()
                    if issue_next:
                        cp(x_hbm.at[sv.at[u + 2]], rows[bp], gs[bp])

                def blkrun(blk, c):
                    r0 = (sid * nblk + blk) * eb
                    pltpu.sync_copy(src_hbm.at[pl.ds(r0, eb)], sv)
                    pltpu.sync_copy(dst_hbm.at[pl.ds(r0, eb)], dv)
                    cp(x_hbm.at[sv.at[0]], rows[0], gs[0])
                    cp(x_hbm.at[sv.at[1]], rows[1], gs[1])
                    unit(0, 0, False, True)
                    unit(1, 1, True, True)
                    unit(2, 2, True, True)

                    def step(i, c2):
                        u = 3 * i + 3
                        unit(u, 0, True, True)
                        unit(u + 1, 1, True, True)
                        unit(u + 2, 2, True, True)
                        return c2

                    lax.fori_loop(0, eb // 3 - 2, step, 0)
                    unit(eb - 5, 0, True, True)
                    unit(eb - 4, 1, True, True)
                    unit(eb - 3, 2, True, True)
                    unit(eb - 2, 0, True, False)
                    unit(eb - 1, 1, True, False)
                    pltpu.make_async_copy(rows[1], acc.at[sidx[1]],
                                          ss[1]).wait()
                    return c

                lax.fori_loop(0, nblk, blkrun, 0)
                plsc.subcore_barrier()
                pltpu.sync_copy(acc.at[pl.ds(sid * rpt, rpt)],
                                out_hbm.at[pl.ds(lo + sid * rpt, rpt)])
                plsc.subcore_barrier()

    return pl.kernel(
        body,
        out_type=[jax.ShapeDtypeStruct((nd_p, D), F32)
                  for (nd_p, _) in jobs],
        mesh=_mesh(),
        scratch_types=(
            [pltpu.VMEM_SHARED((amax, D), F32),
             pltpu.VMEM((32, 32), I32),
             pltpu.VMEM((32, 32), I32)]
            + [pltpu.VMEM((32, D), F32)] * 3
            + [pltpu.VMEM((32,), I32)] * 3
            + [pltpu.VMEM((8, D), F32)]
            + [pltpu.SemaphoreType.DMA] * 7
        ),
    )


def _make_pair_gather(elp):
    per_w = elp // (NC * NS)
    n_g = per_w // 64

    def body(xs_hbm, xm_hbm, li0_hbm, li1_hbm, es_hbm, em_hbm,
             iv, rows, sem):
        cid = lax.axis_index("c")
        sid = lax.axis_index("s")
        w = cid * NS + sid
        r0 = w * n_g

        for (src, idx_hbm, out_hbm) in ((xs_hbm, li0_hbm, es_hbm),
                                        (xm_hbm, li1_hbm, em_hbm)):
            pltpu.sync_copy(idx_hbm.at[pl.ds(r0, n_g)], iv)

            def it(g, c):
                pltpu.async_copy(src.at[iv.at[g]], rows, sem).wait()
                pltpu.sync_copy(rows,
                                out_hbm.at[pl.ds((r0 + g) * 64, 64)])
                return c

            lax.fori_loop(0, n_g, it, 0)

    return pl.kernel(
        body,
        out_type=[jax.ShapeDtypeStruct((elp, D), F32),
                  jax.ShapeDtypeStruct((elp, D), F32)],
        mesh=_mesh(),
        scratch_types=[
            pltpu.VMEM((n_g, 64), I32),
            pltpu.VMEM((64, D), F32),
            pltpu.SemaphoreType.DMA,
        ],
    )


_BLK = 1024


def _gcn_h_body(xm, w1, w2, g1, g2, h1, h2):
    x = xm[...]
    d1 = lax.rsqrt(g1[...] + 1.0)
    d2 = lax.rsqrt(g2[...] + 1.0)
    h1[...] = d1 * jnp.dot(x, w1[...], preferred_element_type=F32)
    h2[...] = d2 * jnp.dot(x, w2[...], preferred_element_type=F32)


def _gcn_h(xm, w1, w2, g1, g2):
    n = xm.shape[0]
    bs_row = pl.BlockSpec((_BLK, D), lambda i: (i, 0))
    bs_w = pl.BlockSpec((D, D), lambda i: (0, 0))
    bs_g = pl.BlockSpec((_BLK, 1), lambda i: (i, 0))
    return pl.pallas_call(
        _gcn_h_body,
        grid=(n // _BLK,),
        in_specs=[bs_row, bs_w, bs_w, bs_g, bs_g],
        out_specs=[bs_row, bs_row],
        out_shape=[jax.ShapeDtypeStruct((n, D), F32)] * 2,
    )(xm, w1, w2, g1, g2)


def _sage_s_body(a, cnt, x, wl, wr, b, o):
    agg = a[...] / jnp.maximum(cnt[...], 1.0)
    o[...] = jax.nn.relu(jnp.dot(agg, wl[...], preferred_element_type=F32)
                         + jnp.dot(x[...], wr[...], preferred_element_type=F32)
                         + b[...])


def _sage_s(acc, cnt, x, wl, wr, b):
    n = x.shape[0]
    bs_row = pl.BlockSpec((_BLK, D), lambda i: (i, 0))
    bs_w = pl.BlockSpec((D, D), lambda i: (0, 0))
    bs_g = pl.BlockSpec((_BLK, 1), lambda i: (i, 0))
    bs_b = pl.BlockSpec((1, D), lambda i: (0, 0))
    return pl.pallas_call(
        _sage_s_body,
        grid=(n // _BLK,),
        in_specs=[bs_row, bs_g, bs_row, bs_w, bs_w, bs_b],
        out_specs=bs_row,
        out_shape=jax.ShapeDtypeStruct((n, D), F32),
    )(acc, cnt, x, wl, wr, b)


def _m_update_body(a1, cnt, x, wl, wr, bb, a2, h1, g1, a3, h2, g2, o):
    agg = a1[...] / jnp.maximum(cnt[...], 1.0)
    t = (jnp.dot(agg, wl[...], preferred_element_type=F32)
         + jnp.dot(x[...], wr[...], preferred_element_type=F32)
         + bb[0:1, :] + bb[1:2, :] + bb[2:3, :])
    d1 = lax.rsqrt(g1[...] + 1.0)
    d2 = lax.rsqrt(g2[...] + 1.0)
    t = t + d1 * (a2[...] + h1[...]) + d2 * (a3[...] + h2[...])
    o[...] = jax.nn.relu(t)


def _m_update(a1, cnt, x, wl, wr, bb, a2, h1, g1, a3, h2, g2):
    n = x.shape[0]
    bs_row = pl.BlockSpec((_BLK, D), lambda i: (i, 0))
    bs_w = pl.BlockSpec((D, D), lambda i: (0, 0))
    bs_g = pl.BlockSpec((_BLK, 1), lambda i: (i, 0))
    bs_b = pl.BlockSpec((3, D), lambda i: (0, 0))
    return pl.pallas_call(
        _m_update_body,
        grid=(n // _BLK,),
        in_specs=[bs_row, bs_g, bs_row, bs_w, bs_w, bs_b,
                  bs_row, bs_row, bs_g, bs_row, bs_row, bs_g],
        out_specs=bs_row,
        out_shape=jax.ShapeDtypeStruct((n, D), F32),
    )(a1, cnt, x, wl, wr, bb, a2, h1, g1, a3, h2, g2)


def _dot_body(a, b, o):
    o[...] = jnp.sum(a[...] * b[...], axis=1, keepdims=True)


def _pair_dot(a, b):
    n = a.shape[0]
    blk = 2048
    bs_row = pl.BlockSpec((blk, D), lambda i: (i, 0))
    bs_o = pl.BlockSpec((blk, 1), lambda i: (i, 0))
    return pl.pallas_call(
        _dot_body,
        grid=(n // blk,),
        in_specs=[bs_row, bs_row],
        out_specs=bs_o,
        out_shape=jax.ShapeDtypeStruct((n, 1), F32),
    )(a, b)


def kernel(params, srna_node_id, mrna_node_id, edge_index_sm,
           edge_index_rev_sm, edge_index_mm, edge_index_rev_mm,
           edge_label_index):
    del srna_node_id, mrna_node_id
    ns = params['srna_emb'].shape[0]
    nm = params['mrna_emb'].shape[0]
    e = edge_index_sm.shape[1]
    el = edge_label_index.shape[1]

    NSP = _ru(ns, NC * NS * L)
    NMP = _ru(nm, NC * 2 * NS * L)
    EP = _ru(e, NS * 32 * 32)
    ELP = _ru(el, NC * NS * 64 * 16)

    xs = jnp.pad(params['srna_emb'].astype(F32), ((0, NSP - ns), (0, 0)))
    xm = jnp.pad(params['mrna_emb'].astype(F32), ((0, NMP - nm), (0, 0)))

    def eprep(ei, pad_dst):
        s = jnp.pad(ei[0].astype(I32), (0, EP - e)).reshape(EP // 32, 32)
        d = jnp.pad(ei[1].astype(I32), (0, EP - e),
                    constant_values=pad_dst).reshape(EP // 32, 32)
        return s, d

    s_sm, d_sm = eprep(edge_index_sm, nm)
    s_rsm, d_rsm = eprep(edge_index_rev_sm, ns)
    s_mm, d_mm = eprep(edge_index_mm, nm)
    s_rmm, d_rmm = eprep(edge_index_rev_mm, nm)

    counts = _make_counts(EP, (NMP, NMP, NSP, NMP))(d_sm, d_mm, d_rsm, d_rmm)
    c_sm = counts[0].reshape(NMP, 1)
    c_mm = counts[1].reshape(NMP, 1)
    c_rsm = counts[2].reshape(NSP, 1)
    c_rmm = counts[3].reshape(NMP, 1)

    seg = _make_segsum(EP, [
        (NSP, 1),
        (NMP, 2),
        (NMP, 2),
        (NMP, 2),
    ])

    for lyr in params['layers']:
        wl_sm, wr_sm, b_sm = lyr['sage_sm']
        wl_ms, wr_ms, b_ms = lyr['sage_ms']
        w_mm, b_mm = lyr['gcn_mm']
        w_rmm, b_rmm = lyr['gcn_rev_mm']

        h1, h2 = _gcn_h(xm, w_mm, w_rmm, c_mm, c_rmm)
        acc_s, acc_m1, acc_m2, acc_m3 = seg(
            xm, xs, h1, h2,
            s_rsm, s_sm, s_mm, s_rmm,
            d_rsm, d_sm, d_mm, d_rmm)
        xs = _sage_s(acc_s, c_rsm, xs, wl_ms, wr_ms, b_ms.reshape(1, D))
        xm = _m_update(acc_m1, c_sm, xm, wl_sm, wr_sm,
                       jnp.stack([b_sm, b_mm, b_rmm]),
                       acc_m2, h1, c_mm, acc_m3, h2, c_rmm)

    li0 = jnp.pad(edge_label_index[0].astype(I32),
                  (0, ELP - el)).reshape(ELP // 64, 64)
    li1 = jnp.pad(edge_label_index[1].astype(I32),
                  (0, ELP - el)).reshape(ELP // 64, 64)
    ef_s, ef_m = _make_pair_gather(ELP)(xs, xm, li0, li1)
    return _pair_dot(ef_s, ef_m)[:el, 0]


# speedup vs baseline: 1.6218x; 1.6218x over previous
import functools

import jax
import jax.numpy as jnp
from jax import lax
from jax.experimental import pallas as pl
from jax.experimental.pallas import tpu as pltpu
from jax.experimental.pallas import tpu_sc as plsc

D = 128
NC, NS, L = 2, 16, 16
G = 128
F32 = jnp.float32
I32 = jnp.int32


@functools.cache
def _mesh():
    return plsc.VectorSubcoreMesh(core_axis_name="c", subcore_axis_name="s",
                                  num_cores=NC, num_subcores=NS)


def _ru(x, m):
    return (x + m - 1) // m * m


def _static_spans(total, step):
    out = []
    off = 0
    while off < total:
        w = min(step, total - off)
        out.append((off, w))
        off += w
    return out


def _make_counts(ep, sizes):
    rpe = ep // 32 // NS
    amax = max(sizes)

    def body(d0, d1, d2, d3, o0, o1, o2, o3, acc, dv, ones, zb, cb, sem):
        del sem
        cid = lax.axis_index("c")
        sid = lax.axis_index("s")
        def init16(i, c):
            zb[0, pl.ds(i * L, L)] = jnp.zeros((L,), F32)
            ones[0, pl.ds(lax.rem(i, jnp.int32(2)) * L, L)] = (
                jnp.ones((L,), F32))
            return c

        lax.fori_loop(0, 1024 // L, init16, 0)

        def job(dst_hbm, out_hbm, n):
            span = n // NS
            base = sid * span
            for (off, w) in _static_spans(span, 1024):
                pltpu.sync_copy(zb.at[0, pl.ds(0, w)],
                                acc.at[pl.ds(base + off, w)])
            plsc.subcore_barrier()
            pltpu.sync_copy(dst_hbm.at[pl.ds(sid * rpe, rpe)], dv)

            def it(j, c):
                pltpu.sync_copy(ones.at[0], acc.at[dv.at[j]], add=True)
                return c

            lax.fori_loop(0, rpe, it, 0)
            plsc.subcore_barrier()
            pltpu.sync_copy(acc.at[pl.ds(base, span)], cb.at[pl.ds(0, span)])
            pltpu.sync_copy(cb.at[pl.ds(0, span)],
                            out_hbm.at[pl.ds(base, span)])
            plsc.subcore_barrier()

        @pl.when(cid == 0)
        def _():
            job(d0, o0, sizes[0])
            job(d1, o1, sizes[1])

        @pl.when(cid == 1)
        def _():
            job(d2, o2, sizes[2])
            job(d3, o3, sizes[3])

    return pl.kernel(
        body,
        out_type=[jax.ShapeDtypeStruct((s,), F32) for s in sizes],
        mesh=_mesh(),
        scratch_types=[
            pltpu.VMEM_SHARED((amax,), F32),
            pltpu.VMEM((rpe, 32), I32),
            pltpu.VMEM((1, 32), F32),
            pltpu.VMEM((1, 1024), F32),
            pltpu.VMEM((amax // NS,), F32),
            pltpu.SemaphoreType.DMA,
        ],
    )


def _make_segsum(ep, jobs):
    gw = 32
    eb = 16
    ept = ep // NS
    nblk = ept // 32 // eb
    amax = max(nd_p // (NC * npc) for (nd_p, npc) in jobs) + 8
    fcap = _ru(ept + 4 * gw + 8, 8)
    trash_slot = fcap - 8
    assert ept % (32 * eb) == 0

    def body(*refs):
        nj = len(jobs)
        xs = refs[0:nj]
        srcs = refs[nj:2 * nj]
        dsts = refs[2 * nj:3 * nj]
        outs = refs[3 * nj:4 * nj]
        (acc, sv, dv, fbuf, r0, r1, r2, x0, x1, x2, u0, u1, u2, zb,
         g0, g1, g2, s0, s1, s2, zs) = refs[4 * nj:]
        rows = (r0, r1, r2)
        sidx = (x0, x1, x2)
        usrc = (u0, u1, u2)
        gs = (g0, g1, g2)
        ss = (s0, s1, s2)
        cid = lax.axis_index("c")
        sid = lax.axis_index("s")

        def zinit(i, c):
            zb[i // (D // L), pl.ds(lax.rem(i, jnp.int32(D // L)) * L, L)] = (
                jnp.zeros((L,), F32))
            return c

        lax.fori_loop(0, 4 * (D // L), zinit, 0)
        cp = pltpu.async_copy

        for j, (nd_p, npc) in enumerate(jobs):
            x_hbm, src_hbm, dst_hbm, out_hbm = xs[j], srcs[j], dsts[j], outs[j]
            chunk = nd_p // (NC * npc)
            rpt = chunk // NS
            nz = rpt // 4
            for cj in range(npc):
                lo = (cid * npc + cj) * chunk
                def ziss(i, c):
                    cp(zb, acc.at[pl.ds(sid * rpt + i * 4, 4)], zs)
                    return c

                def zdrn(i, c):
                    pltpu.make_async_copy(
                        zb, acc.at[pl.ds(sid * rpt, 4)], zs).wait()
                    return c

                lax.fori_loop(0, nz, ziss, 0)
                lax.fori_loop(0, nz, zdrn, 0)
                plsc.subcore_barrier()

                iota = lax.iota(I32, L)

                def blkfilt(blk, ptr):
                    rr = (sid * nblk + blk) * eb
                    pltpu.sync_copy(src_hbm.at[pl.ds(rr, eb)], sv)
                    pltpu.sync_copy(dst_hbm.at[pl.ds(rr, eb)], dv)

                    def filt(k, ptr2):
                        r = k // 2
                        c = lax.rem(k, jnp.int32(2)) * L
                        dd = dv[r, pl.ds(c, L)]
                        ss_ = sv[r, pl.ds(c, L)]
                        m = (dd >= lo) & (dd < lo + chunk)
                        mi = m.astype(I32)
                        p = mi
                        for sh in (1, 2, 4, 8):
                            g = p[jnp.maximum(iota - sh, 0)]
                            p = p + jnp.where(iota >= sh, g, 0)
                        pos = jnp.where(m, ptr2 + (p - mi),
                                        jnp.int32(trash_slot))
                        v = ss_ | ((dd - lo) << 16)
                        plsc.store_scatter(fbuf, [pos], v)
                        return ptr2 + p[L - 1]

                    return lax.fori_loop(0, eb * 2, filt, ptr)

                n = lax.fori_loop(0, nblk, blkfilt, jnp.int32(0))
                vpad = jnp.full((L,), chunk << 16, I32)
                for t in range(4 * gw // L):
                    fbuf[pl.ds(n + t * L, L)] = vpad
                n3 = jnp.maximum(lax.div(n + 3 * gw - 1, jnp.int32(3 * gw)),
                                 jnp.int32(1))

                def unpack(u, b):
                    for k in range(gw // L):
                        vv = fbuf[pl.ds(u * gw + k * L, L)]
                        usrc[b][pl.ds(k * L, L)] = vv & jnp.int32(0xFFFF)
                        sidx[b][pl.ds(k * L, L)] = (
                            lax.shift_right_logical(vv, 16))

                def unit(u, b, wait_prev, issue_next):
                    pltpu.make_async_copy(x_hbm.at[usrc[b]], rows[b],
                                          gs[b]).wait()
                    cp(rows[b], acc.at[sidx[b]], ss[b], add=True)
                    bp = (b + 2) % 3
                    if wait_prev is not None:
                        def _w():
                            pltpu.make_async_copy(rows[bp], acc.at[sidx[bp]],
                                                  ss[bp]).wait()
                        if wait_prev is True:
                            _w()
                        else:
                            pl.when(wait_prev)(_w)
                    if issue_next is not None:
                        def _i():
                            unpack(u + 2, bp)
                            cp(x_hbm.at[usrc[bp]], rows[bp], gs[bp])
                        if issue_next is True:
                            _i()
                        else:
                            pl.when(issue_next)(_i)

                unpack(0, 0)
                cp(x_hbm.at[usrc[0]], rows[0], gs[0])
                unpack(1, 1)
                cp(x_hbm.at[usrc[1]], rows[1], gs[1])

                def step(i, c2):
                    u = 3 * i
                    unit(u, 0, i > 0, True)
                    unit(u + 1, 1, True, i < n3 - 1)
                    unit(u + 2, 2, True, i < n3 - 1)
                    return c2

                lax.fori_loop(0, n3, step, 0)
                pltpu.make_async_copy(rows[2], acc.at[sidx[2]], ss[2]).wait()
                plsc.subcore_barrier()
                pltpu.sync_copy(acc.at[pl.ds(sid * rpt, rpt)],
                                out_hbm.at[pl.ds(lo + sid * rpt, rpt)])
                plsc.subcore_barrier()

    return pl.kernel(
        body,
        out_type=[jax.ShapeDtypeStruct((nd_p, D), F32)
                  for (nd_p, _) in jobs],
        mesh=_mesh(),
        scratch_types=(
            [pltpu.VMEM_SHARED((amax, D), F32),
             pltpu.VMEM((eb, 32), I32),
             pltpu.VMEM((eb, 32), I32),
             pltpu.VMEM((fcap,), I32)]
            + [pltpu.VMEM((gw, D), F32)] * 3
            + [pltpu.VMEM((gw,), I32)] * 3
            + [pltpu.VMEM((gw,), I32)] * 3
            + [pltpu.VMEM((4, D), F32)]
            + [pltpu.SemaphoreType.DMA] * 7
        ),
        compiler_params=pltpu.CompilerParams(needs_layout_passes=False),
    )


def _make_pair_gather(elp):
    per_w = elp // (NC * NS)
    n_g = per_w // 64

    def body(xs_hbm, xm_hbm, li0_hbm, li1_hbm, es_hbm, em_hbm,
             iv, rows, sem):
        cid = lax.axis_index("c")
        sid = lax.axis_index("s")
        w = cid * NS + sid
        r0 = w * n_g

        for (src, idx_hbm, out_hbm) in ((xs_hbm, li0_hbm, es_hbm),
                                        (xm_hbm, li1_hbm, em_hbm)):
            pltpu.sync_copy(idx_hbm.at[pl.ds(r0, n_g)], iv)

            def it(g, c):
                pltpu.async_copy(src.at[iv.at[g]], rows, sem).wait()
                pltpu.sync_copy(rows,
                                out_hbm.at[pl.ds((r0 + g) * 64, 64)])
                return c

            lax.fori_loop(0, n_g, it, 0)

    return pl.kernel(
        body,
        out_type=[jax.ShapeDtypeStruct((elp, D), F32),
                  jax.ShapeDtypeStruct((elp, D), F32)],
        mesh=_mesh(),
        scratch_types=[
            pltpu.VMEM((n_g, 64), I32),
            pltpu.VMEM((64, D), F32),
            pltpu.SemaphoreType.DMA,
        ],
    )


_BLK = 1024


def _gcn_h_body(xm, w1, w2, g1, g2, h1, h2):
    x = xm[...]
    d1 = lax.rsqrt(g1[...] + 1.0)
    d2 = lax.rsqrt(g2[...] + 1.0)
    h1[...] = d1 * jnp.dot(x, w1[...], preferred_element_type=F32)
    h2[...] = d2 * jnp.dot(x, w2[...], preferred_element_type=F32)


def _gcn_h(xm, w1, w2, g1, g2):
    n = xm.shape[0]
    bs_row = pl.BlockSpec((_BLK, D), lambda i: (i, 0))
    bs_w = pl.BlockSpec((D, D), lambda i: (0, 0))
    bs_g = pl.BlockSpec((_BLK, 1), lambda i: (i, 0))
    return pl.pallas_call(
        _gcn_h_body,
        grid=(n // _BLK,),
        in_specs=[bs_row, bs_w, bs_w, bs_g, bs_g],
        out_specs=[bs_row, bs_row],
        out_shape=[jax.ShapeDtypeStruct((n, D), F32)] * 2,
    )(xm, w1, w2, g1, g2)


def _sage_s_body(a, cnt, x, wl, wr, b, o):
    agg = a[...] / jnp.maximum(cnt[...], 1.0)
    o[...] = jax.nn.relu(jnp.dot(agg, wl[...], preferred_element_type=F32)
                         + jnp.dot(x[...], wr[...], preferred_element_type=F32)
                         + b[...])


def _sage_s(acc, cnt, x, wl, wr, b):
    n = x.shape[0]
    bs_row = pl.BlockSpec((_BLK, D), lambda i: (i, 0))
    bs_w = pl.BlockSpec((D, D), lambda i: (0, 0))
    bs_g = pl.BlockSpec((_BLK, 1), lambda i: (i, 0))
    bs_b = pl.BlockSpec((1, D), lambda i: (0, 0))
    return pl.pallas_call(
        _sage_s_body,
        grid=(n // _BLK,),
        in_specs=[bs_row, bs_g, bs_row, bs_w, bs_w, bs_b],
        out_specs=bs_row,
        out_shape=jax.ShapeDtypeStruct((n, D), F32),
    )(acc, cnt, x, wl, wr, b)


def _m_update_body(a1, cnt, x, wl, wr, bb, a2, h1, g1, a3, h2, g2, o):
    agg = a1[...] / jnp.maximum(cnt[...], 1.0)
    t = (jnp.dot(agg, wl[...], preferred_element_type=F32)
         + jnp.dot(x[...], wr[...], preferred_element_type=F32)
         + bb[0:1, :] + bb[1:2, :] + bb[2:3, :])
    d1 = lax.rsqrt(g1[...] + 1.0)
    d2 = lax.rsqrt(g2[...] + 1.0)
    t = t + d1 * (a2[...] + h1[...]) + d2 * (a3[...] + h2[...])
    o[...] = jax.nn.relu(t)


def _m_update(a1, cnt, x, wl, wr, bb, a2, h1, g1, a3, h2, g2):
    n = x.shape[0]
    bs_row = pl.BlockSpec((_BLK, D), lambda i: (i, 0))
    bs_w = pl.BlockSpec((D, D), lambda i: (0, 0))
    bs_g = pl.BlockSpec((_BLK, 1), lambda i: (i, 0))
    bs_b = pl.BlockSpec((3, D), lambda i: (0, 0))
    return pl.pallas_call(
        _m_update_body,
        grid=(n // _BLK,),
        in_specs=[bs_row, bs_g, bs_row, bs_w, bs_w, bs_b,
                  bs_row, bs_row, bs_g, bs_row, bs_row, bs_g],
        out_specs=bs_row,
        out_shape=jax.ShapeDtypeStruct((n, D), F32),
    )(a1, cnt, x, wl, wr, bb, a2, h1, g1, a3, h2, g2)


def _dot_body(a, b, o):
    o[...] = jnp.sum(a[...] * b[...], axis=1, keepdims=True)


def _pair_dot(a, b):
    n = a.shape[0]
    blk = 2048
    bs_row = pl.BlockSpec((blk, D), lambda i: (i, 0))
    bs_o = pl.BlockSpec((blk, 1), lambda i: (i, 0))
    return pl.pallas_call(
        _dot_body,
        grid=(n // blk,),
        in_specs=[bs_row, bs_row],
        out_specs=bs_o,
        out_shape=jax.ShapeDtypeStruct((n, 1), F32),
    )(a, b)


def kernel(params, srna_node_id, mrna_node_id, edge_index_sm,
           edge_index_rev_sm, edge_index_mm, edge_index_rev_mm,
           edge_label_index):
    del srna_node_id, mrna_node_id
    ns = params['srna_emb'].shape[0]
    nm = params['mrna_emb'].shape[0]
    e = edge_index_sm.shape[1]
    el = edge_label_index.shape[1]

    NSP = _ru(ns, NC * NS * L)
    NMP = _ru(nm, NC * 2 * NS * L)
    EP = _ru(e, NS * 32 * 32)
    ELP = _ru(el, NC * NS * 64 * 16)

    xs = jnp.pad(params['srna_emb'].astype(F32), ((0, NSP - ns), (0, 0)))
    xm = jnp.pad(params['mrna_emb'].astype(F32), ((0, NMP - nm), (0, 0)))

    def eprep(ei, pad_dst):
        s = jnp.pad(ei[0].astype(I32), (0, EP - e)).reshape(EP // 32, 32)
        d = jnp.pad(ei[1].astype(I32), (0, EP - e),
                    constant_values=pad_dst).reshape(EP // 32, 32)
        return s, d

    s_sm, d_sm = eprep(edge_index_sm, nm)
    s_rsm, d_rsm = eprep(edge_index_rev_sm, ns)
    s_mm, d_mm = eprep(edge_index_mm, nm)
    s_rmm, d_rmm = eprep(edge_index_rev_mm, nm)

    counts = _make_counts(EP, (NMP, NMP, NSP, NMP))(d_sm, d_mm, d_rsm, d_rmm)
    c_sm = counts[0].reshape(NMP, 1)
    c_mm = counts[1].reshape(NMP, 1)
    c_rsm = counts[2].reshape(NSP, 1)
    c_rmm = counts[3].reshape(NMP, 1)

    seg = _make_segsum(EP, [
        (NSP, 1),
        (NMP, 2),
        (NMP, 2),
        (NMP, 2),
    ])

    for lyr in params['layers']:
        wl_sm, wr_sm, b_sm = lyr['sage_sm']
        wl_ms, wr_ms, b_ms = lyr['sage_ms']
        w_mm, b_mm = lyr['gcn_mm']
        w_rmm, b_rmm = lyr['gcn_rev_mm']

        h1, h2 = _gcn_h(xm, w_mm, w_rmm, c_mm, c_rmm)
        acc_s, acc_m1, acc_m2, acc_m3 = seg(
            xm, xs, h1, h2,
            s_rsm, s_sm, s_mm, s_rmm,
            d_rsm, d_sm, d_mm, d_rmm)
        xs = _sage_s(acc_s, c_rsm, xs, wl_ms, wr_ms, b_ms.reshape(1, D))
        xm = _m_update(acc_m1, c_sm, xm, wl_sm, wr_sm,
                       jnp.stack([b_sm, b_mm, b_rmm]),
                       acc_m2, h1, c_mm, acc_m3, h2, c_rmm)

    li0 = jnp.pad(edge_label_index[0].astype(I32),
                  (0, ELP - el)).reshape(ELP // 64, 64)
    li1 = jnp.pad(edge_label_index[1].astype(I32),
                  (0, ELP - el)).reshape(ELP // 64, 64)
    ef_s, ef_m = _make_pair_gather(ELP)(xs, xm, li0, li1)
    return _pair_dot(ef_s, ef_m)[:el, 0]


# speedup vs baseline: 1.6360x; 1.0087x over previous
import functools

import jax
import jax.numpy as jnp
from jax import lax
from jax.experimental import pallas as pl
from jax.experimental.pallas import tpu as pltpu
from jax.experimental.pallas import tpu_sc as plsc

D = 128
NC, NS, L = 2, 16, 16
G = 128
F32 = jnp.float32
I32 = jnp.int32


@functools.cache
def _mesh():
    return plsc.VectorSubcoreMesh(core_axis_name="c", subcore_axis_name="s",
                                  num_cores=NC, num_subcores=NS)


def _ru(x, m):
    return (x + m - 1) // m * m


def _static_spans(total, step):
    out = []
    off = 0
    while off < total:
        w = min(step, total - off)
        out.append((off, w))
        off += w
    return out


def _make_counts(ep, sizes):
    rpe = ep // 32 // NS
    amax = max(sizes)

    def body(d0, d1, d2, d3, o0, o1, o2, o3, acc, dv, ones, zb, cb, sem):
        del sem
        cid = lax.axis_index("c")
        sid = lax.axis_index("s")
        def init16(i, c):
            zb[0, pl.ds(i * L, L)] = jnp.zeros((L,), F32)
            ones[0, pl.ds(lax.rem(i, jnp.int32(2)) * L, L)] = (
                jnp.ones((L,), F32))
            return c

        lax.fori_loop(0, 1024 // L, init16, 0)

        def job(dst_hbm, out_hbm, n):
            span = n // NS
            base = sid * span
            for (off, w) in _static_spans(span, 1024):
                pltpu.sync_copy(zb.at[0, pl.ds(0, w)],
                                acc.at[pl.ds(base + off, w)])
            plsc.subcore_barrier()
            pltpu.sync_copy(dst_hbm.at[pl.ds(sid * rpe, rpe)], dv)

            def it(j, c):
                pltpu.sync_copy(ones.at[0], acc.at[dv.at[j]], add=True)
                return c

            lax.fori_loop(0, rpe, it, 0)
            plsc.subcore_barrier()
            pltpu.sync_copy(acc.at[pl.ds(base, span)], cb.at[pl.ds(0, span)])
            pltpu.sync_copy(cb.at[pl.ds(0, span)],
                            out_hbm.at[pl.ds(base, span)])
            plsc.subcore_barrier()

        @pl.when(cid == 0)
        def _():
            job(d0, o0, sizes[0])
            job(d1, o1, sizes[1])

        @pl.when(cid == 1)
        def _():
            job(d2, o2, sizes[2])
            job(d3, o3, sizes[3])

    return pl.kernel(
        body,
        out_type=[jax.ShapeDtypeStruct((s,), F32) for s in sizes],
        mesh=_mesh(),
        scratch_types=[
            pltpu.VMEM_SHARED((amax,), F32),
            pltpu.VMEM((rpe, 32), I32),
            pltpu.VMEM((1, 32), F32),
            pltpu.VMEM((1, 1024), F32),
            pltpu.VMEM((amax // NS,), F32),
            pltpu.SemaphoreType.DMA,
        ],
    )


def _make_segsum(ep, jobs):
    gw = 32
    eb = 16
    ept = ep // NS
    nblk = ept // 32 // eb
    amax = max(nd_p // (NC * npc) for (nd_p, npc) in jobs) + 8
    fcap = _ru(ept + 4 * gw + 8, 8)
    trash_slot = fcap - 8
    assert ept % (32 * eb) == 0

    def body(*refs):
        nj = len(jobs)
        xs = refs[0:nj]
        srcs = refs[nj:2 * nj]
        dsts = refs[2 * nj:3 * nj]
        outs = refs[3 * nj:4 * nj]
        (acc, sv, dv, fbuf, r0, r1, r2, x0, x1, x2, u0, u1, u2, zb,
         g0, g1, g2, s0, s1, s2, zs) = refs[4 * nj:]
        rows = (r0, r1, r2)
        sidx = (x0, x1, x2)
        usrc = (u0, u1, u2)
        gs = (g0, g1, g2)
        ss = (s0, s1, s2)
        cid = lax.axis_index("c")
        sid = lax.axis_index("s")

        def zinit(i, c):
            zb[i // (D // L), pl.ds(lax.rem(i, jnp.int32(D // L)) * L, L)] = (
                jnp.zeros((L,), F32))
            return c

        lax.fori_loop(0, 4 * (D // L), zinit, 0)
        cp = pltpu.async_copy

        for j, (nd_p, npc) in enumerate(jobs):
            x_hbm, src_hbm, dst_hbm, out_hbm = xs[j], srcs[j], dsts[j], outs[j]
            chunk = nd_p // (NC * npc)
            rpt = chunk // NS
            nz = rpt // 4
            for cj in range(npc):
                lo = (cid * npc + cj) * chunk
                def ziss(i, c):
                    cp(zb, acc.at[pl.ds(sid * rpt + i * 4, 4)], zs)
                    return c

                def zdrn(i, c):
                    pltpu.make_async_copy(
                        zb, acc.at[pl.ds(sid * rpt, 4)], zs).wait()
                    return c

                lax.fori_loop(0, nz, ziss, 0)
                lax.fori_loop(0, nz, zdrn, 0)
                plsc.subcore_barrier()

                iota = lax.iota(I32, L)

                def blkfilt(blk, ptr):
                    rr = (sid * nblk + blk) * eb
                    pltpu.sync_copy(src_hbm.at[pl.ds(rr, eb)], sv)
                    pltpu.sync_copy(dst_hbm.at[pl.ds(rr, eb)], dv)

                    def filt(r, ptr2):
                        res = []
                        for c in (0, L):
                            dd = dv[r, pl.ds(c, L)]
                            ss_ = sv[r, pl.ds(c, L)]
                            m = (dd >= lo) & (dd < lo + chunk)
                            mi = m.astype(I32)
                            p = mi
                            for sh in (1, 2, 4, 8):
                                g = p[jnp.maximum(iota - sh, 0)]
                                p = p + jnp.where(iota >= sh, g, 0)
                            v = ss_ | ((dd - lo) << 16)
                            res.append((m, p - mi, p[L - 1], v))
                        m0, e0, c0, v0 = res[0]
                        m1, e1, c1, v1 = res[1]
                        pos0 = jnp.where(m0, ptr2 + e0, jnp.int32(trash_slot))
                        plsc.store_scatter(fbuf, [pos0], v0)
                        pos1 = jnp.where(m1, ptr2 + c0 + e1,
                                         jnp.int32(trash_slot))
                        plsc.store_scatter(fbuf, [pos1], v1)
                        return ptr2 + c0 + c1

                    return lax.fori_loop(0, eb, filt, ptr)

                n = lax.fori_loop(0, nblk, blkfilt, jnp.int32(0))
                vpad = jnp.full((L,), chunk << 16, I32)
                for t in range(4 * gw // L):
                    fbuf[pl.ds(n + t * L, L)] = vpad
                n3 = jnp.maximum(lax.div(n + 3 * gw - 1, jnp.int32(3 * gw)),
                                 jnp.int32(1))

                def unpack(u, b):
                    for k in range(gw // L):
                        vv = fbuf[pl.ds(u * gw + k * L, L)]
                        usrc[b][pl.ds(k * L, L)] = vv & jnp.int32(0xFFFF)
                        sidx[b][pl.ds(k * L, L)] = (
                            lax.shift_right_logical(vv, 16))

                def unit(u, b, wait_prev, issue_next):
                    pltpu.make_async_copy(x_hbm.at[usrc[b]], rows[b],
                                          gs[b]).wait()
                    cp(rows[b], acc.at[sidx[b]], ss[b], add=True)
                    bp = (b + 2) % 3
                    if wait_prev is not None:
                        def _w():
                            pltpu.make_async_copy(rows[bp], acc.at[sidx[bp]],
                                                  ss[bp]).wait()
                        if wait_prev is True:
                            _w()
                        else:
                            pl.when(wait_prev)(_w)
                    if issue_next is not None:
                        def _i():
                            unpack(u + 2, bp)
                            cp(x_hbm.at[usrc[bp]], rows[bp], gs[bp])
                        if issue_next is True:
                            _i()
                        else:
                            pl.when(issue_next)(_i)

                unpack(0, 0)
                cp(x_hbm.at[usrc[0]], rows[0], gs[0])
                unpack(1, 1)
                cp(x_hbm.at[usrc[1]], rows[1], gs[1])

                def step(i, c2):
                    u = 3 * i
                    unit(u, 0, i > 0, True)
                    unit(u + 1, 1, True, i < n3 - 1)
                    unit(u + 2, 2, True, i < n3 - 1)
                    return c2

                lax.fori_loop(0, n3, step, 0)
                pltpu.make_async_copy(rows[2], acc.at[sidx[2]], ss[2]).wait()
                plsc.subcore_barrier()
                pltpu.sync_copy(acc.at[pl.ds(sid * rpt, rpt)],
                                out_hbm.at[pl.ds(lo + sid * rpt, rpt)])
                plsc.subcore_barrier()

    return pl.kernel(
        body,
        out_type=[jax.ShapeDtypeStruct((nd_p, D), F32)
                  for (nd_p, _) in jobs],
        mesh=_mesh(),
        scratch_types=(
            [pltpu.VMEM_SHARED((amax, D), F32),
             pltpu.VMEM((eb, 32), I32),
             pltpu.VMEM((eb, 32), I32),
             pltpu.VMEM((fcap,), I32)]
            + [pltpu.VMEM((gw, D), F32)] * 3
            + [pltpu.VMEM((gw,), I32)] * 3
            + [pltpu.VMEM((gw,), I32)] * 3
            + [pltpu.VMEM((4, D), F32)]
            + [pltpu.SemaphoreType.DMA] * 7
        ),
        compiler_params=pltpu.CompilerParams(needs_layout_passes=False),
    )


def _make_pair_gather(elp):
    per_w = elp // (NC * NS)
    n_g = per_w // 64

    def body(xs_hbm, xm_hbm, li0_hbm, li1_hbm, es_hbm, em_hbm,
             iv, rowsa, rowsb, sema, semb):
        cid = lax.axis_index("c")
        sid = lax.axis_index("s")
        w = cid * NS + sid
        r0 = w * n_g
        rows = (rowsa, rowsb)
        sems = (sema, semb)

        for (src, idx_hbm, out_hbm) in ((xs_hbm, li0_hbm, es_hbm),
                                        (xm_hbm, li1_hbm, em_hbm)):
            pltpu.sync_copy(idx_hbm.at[pl.ds(r0, n_g)], iv)
            pltpu.async_copy(src.at[iv.at[0]], rows[0], sems[0])

            def it(i, c):
                g = 2 * i
                pltpu.make_async_copy(src.at[iv.at[0]], rows[0],
                                      sems[0]).wait()
                pltpu.async_copy(src.at[iv.at[g + 1]], rows[1], sems[1])
                pltpu.sync_copy(rows[0],
                                out_hbm.at[pl.ds((r0 + g) * 64, 64)])
                pltpu.make_async_copy(src.at[iv.at[0]], rows[1],
                                      sems[1]).wait()

                @pl.when(i < n_g // 2 - 1)
                def _():
                    pltpu.async_copy(src.at[iv.at[g + 2]], rows[0], sems[0])

                pltpu.sync_copy(rows[1],
                                out_hbm.at[pl.ds((r0 + g + 1) * 64, 64)])
                return c

            lax.fori_loop(0, n_g // 2, it, 0)

    return pl.kernel(
        body,
        out_type=[jax.ShapeDtypeStruct((elp, D), F32),
                  jax.ShapeDtypeStruct((elp, D), F32)],
        mesh=_mesh(),
        scratch_types=[
            pltpu.VMEM((n_g, 64), I32),
            pltpu.VMEM((64, D), F32),
            pltpu.VMEM((64, D), F32),
            pltpu.SemaphoreType.DMA,
            pltpu.SemaphoreType.DMA,
        ],
    )


_BLK = 1024


def _gcn_h_body(xm, w1, w2, g1, g2, h1, h2):
    x = xm[...]
    d1 = lax.rsqrt(g1[...] + 1.0)
    d2 = lax.rsqrt(g2[...] + 1.0)
    h1[...] = d1 * jnp.dot(x, w1[...], preferred_element_type=F32)
    h2[...] = d2 * jnp.dot(x, w2[...], preferred_element_type=F32)


def _gcn_h(xm, w1, w2, g1, g2):
    n = xm.shape[0]
    bs_row = pl.BlockSpec((_BLK, D), lambda i: (i, 0))
    bs_w = pl.BlockSpec((D, D), lambda i: (0, 0))
    bs_g = pl.BlockSpec((_BLK, 1), lambda i: (i, 0))
    return pl.pallas_call(
        _gcn_h_body,
        grid=(n // _BLK,),
        in_specs=[bs_row, bs_w, bs_w, bs_g, bs_g],
        out_specs=[bs_row, bs_row],
        out_shape=[jax.ShapeDtypeStruct((n, D), F32)] * 2,
    )(xm, w1, w2, g1, g2)


def _sage_s_body(a, cnt, x, wl, wr, b, o):
    agg = a[...] / jnp.maximum(cnt[...], 1.0)
    o[...] = jax.nn.relu(jnp.dot(agg, wl[...], preferred_element_type=F32)
                         + jnp.dot(x[...], wr[...], preferred_element_type=F32)
                         + b[...])


def _sage_s(acc, cnt, x, wl, wr, b):
    n = x.shape[0]
    bs_row = pl.BlockSpec((_BLK, D), lambda i: (i, 0))
    bs_w = pl.BlockSpec((D, D), lambda i: (0, 0))
    bs_g = pl.BlockSpec((_BLK, 1), lambda i: (i, 0))
    bs_b = pl.BlockSpec((1, D), lambda i: (0, 0))
    return pl.pallas_call(
        _sage_s_body,
        grid=(n // _BLK,),
        in_specs=[bs_row, bs_g, bs_row, bs_w, bs_w, bs_b],
        out_specs=bs_row,
        out_shape=jax.ShapeDtypeStruct((n, D), F32),
    )(acc, cnt, x, wl, wr, b)


def _m_update_body(a1, cnt, x, wl, wr, bb, a2, h1, g1, a3, h2, g2, o):
    agg = a1[...] / jnp.maximum(cnt[...], 1.0)
    t = (jnp.dot(agg, wl[...], preferred_element_type=F32)
         + jnp.dot(x[...], wr[...], preferred_element_type=F32)
         + bb[0:1, :] + bb[1:2, :] + bb[2:3, :])
    d1 = lax.rsqrt(g1[...] + 1.0)
    d2 = lax.rsqrt(g2[...] + 1.0)
    t = t + d1 * (a2[...] + h1[...]) + d2 * (a3[...] + h2[...])
    o[...] = jax.nn.relu(t)


def _m_update(a1, cnt, x, wl, wr, bb, a2, h1, g1, a3, h2, g2):
    n = x.shape[0]
    bs_row = pl.BlockSpec((_BLK, D), lambda i: (i, 0))
    bs_w = pl.BlockSpec((D, D), lambda i: (0, 0))
    bs_g = pl.BlockSpec((_BLK, 1), lambda i: (i, 0))
    bs_b = pl.BlockSpec((3, D), lambda i: (0, 0))
    return pl.pallas_call(
        _m_update_body,
        grid=(n // _BLK,),
        in_specs=[bs_row, bs_g, bs_row, bs_w, bs_w, bs_b,
                  bs_row, bs_row, bs_g, bs_row, bs_row, bs_g],
        out_specs=bs_row,
        out_shape=jax.ShapeDtypeStruct((n, D), F32),
    )(a1, cnt, x, wl, wr, bb, a2, h1, g1, a3, h2, g2)


def _dot_body(a, b, o):
    o[...] = jnp.sum(a[...] * b[...], axis=1, keepdims=True)


def _pair_dot(a, b):
    n = a.shape[0]
    blk = 2048
    bs_row = pl.BlockSpec((blk, D), lambda i: (i, 0))
    bs_o = pl.BlockSpec((blk, 1), lambda i: (i, 0))
    return pl.pallas_call(
        _dot_body,
        grid=(n // blk,),
        in_specs=[bs_row, bs_row],
        out_specs=bs_o,
        out_shape=jax.ShapeDtypeStruct((n, 1), F32),
    )(a, b)


def kernel(params, srna_node_id, mrna_node_id, edge_index_sm,
           edge_index_rev_sm, edge_index_mm, edge_index_rev_mm,
           edge_label_index):
    del srna_node_id, mrna_node_id
    ns = params['srna_emb'].shape[0]
    nm = params['mrna_emb'].shape[0]
    e = edge_index_sm.shape[1]
    el = edge_label_index.shape[1]

    NSP = _ru(ns, NC * NS * L)
    NMP = _ru(nm, NC * 2 * NS * L)
    EP = _ru(e, NS * 32 * 32)
    ELP = _ru(el, NC * NS * 64 * 16)

    xs = jnp.pad(params['srna_emb'].astype(F32), ((0, NSP - ns), (0, 0)))
    xm = jnp.pad(params['mrna_emb'].astype(F32), ((0, NMP - nm), (0, 0)))

    def eprep(ei, pad_dst):
        s = jnp.pad(ei[0].astype(I32), (0, EP - e)).reshape(EP // 32, 32)
        d = jnp.pad(ei[1].astype(I32), (0, EP - e),
                    constant_values=pad_dst).reshape(EP // 32, 32)
        return s, d

    s_sm, d_sm = eprep(edge_index_sm, nm)
    s_rsm, d_rsm = eprep(edge_index_rev_sm, ns)
    s_mm, d_mm = eprep(edge_index_mm, nm)
    s_rmm, d_rmm = eprep(edge_index_rev_mm, nm)

    counts = _make_counts(EP, (NMP, NMP, NSP, NMP))(d_sm, d_mm, d_rsm, d_rmm)
    c_sm = counts[0].reshape(NMP, 1)
    c_mm = counts[1].reshape(NMP, 1)
    c_rsm = counts[2].reshape(NSP, 1)
    c_rmm = counts[3].reshape(NMP, 1)

    seg = _make_segsum(EP, [
        (NSP, 1),
        (NMP, 2),
        (NMP, 2),
        (NMP, 2),
    ])

    for lyr in params['layers']:
        wl_sm, wr_sm, b_sm = lyr['sage_sm']
        wl_ms, wr_ms, b_ms = lyr['sage_ms']
        w_mm, b_mm = lyr['gcn_mm']
        w_rmm, b_rmm = lyr['gcn_rev_mm']

        h1, h2 = _gcn_h(xm, w_mm, w_rmm, c_mm, c_rmm)
        acc_s, acc_m1, acc_m2, acc_m3 = seg(
            xm, xs, h1, h2,
            s_rsm, s_sm, s_mm, s_rmm,
            d_rsm, d_sm, d_mm, d_rmm)
        xs = _sage_s(acc_s, c_rsm, xs, wl_ms, wr_ms, b_ms.reshape(1, D))
        xm = _m_update(acc_m1, c_sm, xm, wl_sm, wr_sm,
                       jnp.stack([b_sm, b_mm, b_rmm]),
                       acc_m2, h1, c_mm, acc_m3, h2, c_rmm)

    li0 = jnp.pad(edge_label_index[0].astype(I32),
                  (0, ELP - el)).reshape(ELP // 64, 64)
    li1 = jnp.pad(edge_label_index[1].astype(I32),
                  (0, ELP - el)).reshape(ELP // 64, 64)
    ef_s, ef_m = _make_pair_gather(ELP)(xs, xm, li0, li1)
    return _pair_dot(ef_s, ef_m)[:el, 0]


# speedup vs baseline: 1.6361x; 1.0001x over previous
import functools

import jax
import jax.numpy as jnp
from jax import lax
from jax.experimental import pallas as pl
from jax.experimental.pallas import tpu as pltpu
from jax.experimental.pallas import tpu_sc as plsc

D = 128
NC, NS, L = 2, 16, 16
F32 = jnp.float32
I32 = jnp.int32


@functools.cache
def _mesh():
    return plsc.VectorSubcoreMesh(core_axis_name="c", subcore_axis_name="s",
                                  num_cores=NC, num_subcores=NS)


def _ru(x, m):
    return (x + m - 1) // m * m


def _static_spans(total, step):
    out = []
    off = 0
    while off < total:
        w = min(step, total - off)
        out.append((off, w))
        off += w
    return out


def _make_counts(ep, sizes):
    rpe = ep // 32 // NS
    amax = max(sizes)

    def body(d0, d1, d2, d3, o0, o1, o2, o3, acc, dv, ones, zb, cb, sem):
        del sem
        cid = lax.axis_index("c")
        sid = lax.axis_index("s")
        def init16(i, c):
            zb[0, pl.ds(i * L, L)] = jnp.zeros((L,), F32)
            ones[0, pl.ds(lax.rem(i, jnp.int32(2)) * L, L)] = (
                jnp.ones((L,), F32))
            return c

        lax.fori_loop(0, 1024 // L, init16, 0)

        def job(dst_hbm, out_hbm, n):
            span = n // NS
            base = sid * span
            for (off, w) in _static_spans(span, 1024):
                pltpu.sync_copy(zb.at[0, pl.ds(0, w)],
                                acc.at[pl.ds(base + off, w)])
            plsc.subcore_barrier()
            pltpu.sync_copy(dst_hbm.at[pl.ds(sid * rpe, rpe)], dv)

            def it(j, c):
                pltpu.sync_copy(ones.at[0], acc.at[dv.at[j]], add=True)
                return c

            lax.fori_loop(0, rpe, it, 0)
            plsc.subcore_barrier()
            pltpu.sync_copy(acc.at[pl.ds(base, span)], cb.at[pl.ds(0, span)])
            pltpu.sync_copy(cb.at[pl.ds(0, span)],
                            out_hbm.at[pl.ds(base, span)])
            plsc.subcore_barrier()

        @pl.when(cid == 0)
        def _():
            job(d0, o0, sizes[0])
            job(d1, o1, sizes[1])

        @pl.when(cid == 1)
        def _():
            job(d2, o2, sizes[2])
            job(d3, o3, sizes[3])

    return pl.kernel(
        body,
        out_type=[jax.ShapeDtypeStruct((s,), F32) for s in sizes],
        mesh=_mesh(),
        scratch_types=[
            pltpu.VMEM_SHARED((amax,), F32),
            pltpu.VMEM((rpe, 32), I32),
            pltpu.VMEM((1, 32), F32),
            pltpu.VMEM((1, 1024), F32),
            pltpu.VMEM((amax // NS,), F32),
            pltpu.SemaphoreType.DMA,
        ],
    )


def _make_segsum(ep, jobs):
    gw = 32
    eb = 16
    ept = ep // NS
    nblk = ept // 32 // eb
    amax = max(nd_p // (NC * npc) for (nd_p, npc) in jobs) + 8
    fcap = _ru(ept + 4 * gw + 8, 8)
    trash_slot = fcap - 8
    assert ept % (32 * eb) == 0

    def body(*refs):
        nj = len(jobs)
        xs = refs[0:nj]
        srcs = refs[nj:2 * nj]
        dsts = refs[2 * nj:3 * nj]
        outs = refs[3 * nj:4 * nj]
        (acc, sv, dv, fbuf, r0, r1, r2, x0, x1, x2, u0, u1, u2, zb,
         g0, g1, g2, s0, s1, s2, zs) = refs[4 * nj:]
        rows = (r0, r1, r2)
        sidx = (x0, x1, x2)
        usrc = (u0, u1, u2)
        gs = (g0, g1, g2)
        ss = (s0, s1, s2)
        cid = lax.axis_index("c")
        sid = lax.axis_index("s")

        def zinit(i, c):
            zb[i // (D // L), pl.ds(lax.rem(i, jnp.int32(D // L)) * L, L)] = (
                jnp.zeros((L,), F32))
            return c

        lax.fori_loop(0, 4 * (D // L), zinit, 0)
        cp = pltpu.async_copy

        for j, (nd_p, npc) in enumerate(jobs):
            x_hbm, src_hbm, dst_hbm, out_hbm = xs[j], srcs[j], dsts[j], outs[j]
            chunk = nd_p // (NC * npc)
            rpt = chunk // NS
            nz = rpt // 4
            for cj in range(npc):
                lo = (cid * npc + cj) * chunk
                def ziss(i, c):
                    cp(zb, acc.at[pl.ds(sid * rpt + i * 4, 4)], zs)
                    return c

                def zdrn(i, c):
                    pltpu.make_async_copy(
                        zb, acc.at[pl.ds(sid * rpt, 4)], zs).wait()
                    return c

                lax.fori_loop(0, nz, ziss, 0)
                lax.fori_loop(0, nz, zdrn, 0)
                plsc.subcore_barrier()

                iota = lax.iota(I32, L)

                def blkfilt(blk, ptr):
                    rr = (sid * nblk + blk) * eb
                    pltpu.sync_copy(src_hbm.at[pl.ds(rr, eb)], sv)
                    pltpu.sync_copy(dst_hbm.at[pl.ds(rr, eb)], dv)

                    def filt(r, ptr2):
                        res = []
                        for c in (0, L):
                            dd = dv[r, pl.ds(c, L)]
                            ss_ = sv[r, pl.ds(c, L)]
                            m = (dd >= lo) & (dd < lo + chunk)
                            mi = m.astype(I32)
                            p = mi
                            for sh in (1, 2, 4, 8):
                                g = p[jnp.maximum(iota - sh, 0)]
                                p = p + jnp.where(iota >= sh, g, 0)
                            v = ss_ | ((dd - lo) << 16)
                            res.append((m, p - mi, p[L - 1], v))
                        m0, e0, c0, v0 = res[0]
                        m1, e1, c1, v1 = res[1]
                        pos0 = jnp.where(m0, ptr2 + e0, jnp.int32(trash_slot))
                        plsc.store_scatter(fbuf, [pos0], v0)
                        pos1 = jnp.where(m1, ptr2 + c0 + e1,
                                         jnp.int32(trash_slot))
                        plsc.store_scatter(fbuf, [pos1], v1)
                        return ptr2 + c0 + c1

                    return lax.fori_loop(0, eb, filt, ptr)

                n = lax.fori_loop(0, nblk, blkfilt, jnp.int32(0))
                vpad = jnp.full((L,), chunk << 16, I32)
                for t in range(4 * gw // L):
                    fbuf[pl.ds(n + t * L, L)] = vpad
                n3 = jnp.maximum(lax.div(n + 3 * gw - 1, jnp.int32(3 * gw)),
                                 jnp.int32(1))

                def unpack(u, b):
                    for k in range(gw // L):
                        vv = fbuf[pl.ds(u * gw + k * L, L)]
                        usrc[b][pl.ds(k * L, L)] = vv & jnp.int32(0xFFFF)
                        sidx[b][pl.ds(k * L, L)] = (
                            lax.shift_right_logical(vv, 16))

                def unit(u, b, wait_prev, issue_next):
                    pltpu.make_async_copy(x_hbm.at[usrc[b]], rows[b],
                                          gs[b]).wait()
                    cp(rows[b], acc.at[sidx[b]], ss[b], add=True)
                    bp = (b + 2) % 3
                    if wait_prev is not None:
                        def _w():
                            pltpu.make_async_copy(rows[bp], acc.at[sidx[bp]],
                                                  ss[bp]).wait()
                        if wait_prev is True:
                            _w()
                        else:
                            pl.when(wait_prev)(_w)
                    if issue_next is not None:
                        def _i():
                            unpack(u + 2, bp)
                            cp(x_hbm.at[usrc[bp]], rows[bp], gs[bp])
                        if issue_next is True:
                            _i()
                        else:
                            pl.when(issue_next)(_i)

                unpack(0, 0)
                cp(x_hbm.at[usrc[0]], rows[0], gs[0])
                unpack(1, 1)
                cp(x_hbm.at[usrc[1]], rows[1], gs[1])

                def step(i, c2):
                    u = 3 * i
                    unit(u, 0, i > 0, True)
                    unit(u + 1, 1, True, i < n3 - 1)
                    unit(u + 2, 2, True, i < n3 - 1)
                    return c2

                lax.fori_loop(0, n3, step, 0)
                pltpu.make_async_copy(rows[2], acc.at[sidx[2]], ss[2]).wait()
                plsc.subcore_barrier()
                pltpu.sync_copy(acc.at[pl.ds(sid * rpt, rpt)],
                                out_hbm.at[pl.ds(lo + sid * rpt, rpt)])
                plsc.subcore_barrier()

    return pl.kernel(
        body,
        out_type=[jax.ShapeDtypeStruct((nd_p, D), F32)
                  for (nd_p, _) in jobs],
        mesh=_mesh(),
        scratch_types=(
            [pltpu.VMEM_SHARED((amax, D), F32),
             pltpu.VMEM((eb, 32), I32),
             pltpu.VMEM((eb, 32), I32),
             pltpu.VMEM((fcap,), I32)]
            + [pltpu.VMEM((gw, D), F32)] * 3
            + [pltpu.VMEM((gw,), I32)] * 3
            + [pltpu.VMEM((gw,), I32)] * 3
            + [pltpu.VMEM((4, D), F32)]
            + [pltpu.SemaphoreType.DMA] * 7
        ),
        compiler_params=pltpu.CompilerParams(needs_layout_passes=False),
    )


def _make_pair_gather(elp):
    per_w = elp // (NC * NS)
    n_g = per_w // 64

    def body(xs_hbm, xm_hbm, li0_hbm, li1_hbm, es_hbm, em_hbm,
             iv, rowsa, rowsb, sema, semb):
        cid = lax.axis_index("c")
        sid = lax.axis_index("s")
        w = cid * NS + sid
        r0 = w * n_g
        rows = (rowsa, rowsb)
        sems = (sema, semb)

        for (src, idx_hbm, out_hbm) in ((xs_hbm, li0_hbm, es_hbm),
                                        (xm_hbm, li1_hbm, em_hbm)):
            pltpu.sync_copy(idx_hbm.at[pl.ds(r0, n_g)], iv)
            pltpu.async_copy(src.at[iv.at[0]], rows[0], sems[0])

            def it(i, c):
                g = 2 * i
                pltpu.make_async_copy(src.at[iv.at[0]], rows[0],
                                      sems[0]).wait()
                pltpu.async_copy(src.at[iv.at[g + 1]], rows[1], sems[1])
                pltpu.sync_copy(rows[0],
                                out_hbm.at[pl.ds((r0 + g) * 64, 64)])
                pltpu.make_async_copy(src.at[iv.at[0]], rows[1],
                                      sems[1]).wait()

                @pl.when(i < n_g // 2 - 1)
                def _():
                    pltpu.async_copy(src.at[iv.at[g + 2]], rows[0], sems[0])

                pltpu.sync_copy(rows[1],
                                out_hbm.at[pl.ds((r0 + g + 1) * 64, 64)])
                return c

            lax.fori_loop(0, n_g // 2, it, 0)

    return pl.kernel(
        body,
        out_type=[jax.ShapeDtypeStruct((elp, D), F32),
                  jax.ShapeDtypeStruct((elp, D), F32)],
        mesh=_mesh(),
        scratch_types=[
            pltpu.VMEM((n_g, 64), I32),
            pltpu.VMEM((64, D), F32),
            pltpu.VMEM((64, D), F32),
            pltpu.SemaphoreType.DMA,
            pltpu.SemaphoreType.DMA,
        ],
    )


_BLK = 1024


def _gcn_h_body(xm, w1, w2, g1, g2, h1, h2):
    x = xm[...]
    d1 = lax.rsqrt(g1[...] + 1.0)
    d2 = lax.rsqrt(g2[...] + 1.0)
    h1[...] = d1 * jnp.dot(x, w1[...], preferred_element_type=F32)
    h2[...] = d2 * jnp.dot(x, w2[...], preferred_element_type=F32)


def _gcn_h(xm, w1, w2, g1, g2):
    n = xm.shape[0]
    bs_row = pl.BlockSpec((_BLK, D), lambda i: (i, 0))
    bs_w = pl.BlockSpec((D, D), lambda i: (0, 0))
    bs_g = pl.BlockSpec((_BLK, 1), lambda i: (i, 0))
    return pl.pallas_call(
        _gcn_h_body,
        grid=(n // _BLK,),
        in_specs=[bs_row, bs_w, bs_w, bs_g, bs_g],
        out_specs=[bs_row, bs_row],
        out_shape=[jax.ShapeDtypeStruct((n, D), F32)] * 2,
    )(xm, w1, w2, g1, g2)


def _sage_s_body(a, cnt, x, wl, wr, b, o):
    agg = a[...] / jnp.maximum(cnt[...], 1.0)
    o[...] = jax.nn.relu(jnp.dot(agg, wl[...], preferred_element_type=F32)
                         + jnp.dot(x[...], wr[...], preferred_element_type=F32)
                         + b[...])


def _sage_s(acc, cnt, x, wl, wr, b):
    n = x.shape[0]
    bs_row = pl.BlockSpec((_BLK, D), lambda i: (i, 0))
    bs_w = pl.BlockSpec((D, D), lambda i: (0, 0))
    bs_g = pl.BlockSpec((_BLK, 1), lambda i: (i, 0))
    bs_b = pl.BlockSpec((1, D), lambda i: (0, 0))
    return pl.pallas_call(
        _sage_s_body,
        grid=(n // _BLK,),
        in_specs=[bs_row, bs_g, bs_row, bs_w, bs_w, bs_b],
        out_specs=bs_row,
        out_shape=jax.ShapeDtypeStruct((n, D), F32),
    )(acc, cnt, x, wl, wr, b)


def _m_update_body(a1, cnt, x, wl, wr, bb, a2, h1, g1, a3, h2, g2, o):
    agg = a1[...] / jnp.maximum(cnt[...], 1.0)
    t = (jnp.dot(agg, wl[...], preferred_element_type=F32)
         + jnp.dot(x[...], wr[...], preferred_element_type=F32)
         + bb[0:1, :] + bb[1:2, :] + bb[2:3, :])
    d1 = lax.rsqrt(g1[...] + 1.0)
    d2 = lax.rsqrt(g2[...] + 1.0)
    t = t + d1 * (a2[...] + h1[...]) + d2 * (a3[...] + h2[...])
    o[...] = jax.nn.relu(t)


def _m_update(a1, cnt, x, wl, wr, bb, a2, h1, g1, a3, h2, g2):
    n = x.shape[0]
    bs_row = pl.BlockSpec((_BLK, D), lambda i: (i, 0))
    bs_w = pl.BlockSpec((D, D), lambda i: (0, 0))
    bs_g = pl.BlockSpec((_BLK, 1), lambda i: (i, 0))
    bs_b = pl.BlockSpec((3, D), lambda i: (0, 0))
    return pl.pallas_call(
        _m_update_body,
        grid=(n // _BLK,),
        in_specs=[bs_row, bs_g, bs_row, bs_w, bs_w, bs_b,
                  bs_row, bs_row, bs_g, bs_row, bs_row, bs_g],
        out_specs=bs_row,
        out_shape=jax.ShapeDtypeStruct((n, D), F32),
    )(a1, cnt, x, wl, wr, bb, a2, h1, g1, a3, h2, g2)


def _dot_body(a, b, o):
    o[...] = jnp.sum(a[...] * b[...], axis=1, keepdims=True)


def _pair_dot(a, b):
    n = a.shape[0]
    blk = 2048
    bs_row = pl.BlockSpec((blk, D), lambda i: (i, 0))
    bs_o = pl.BlockSpec((blk, 1), lambda i: (i, 0))
    return pl.pallas_call(
        _dot_body,
        grid=(n // blk,),
        in_specs=[bs_row, bs_row],
        out_specs=bs_o,
        out_shape=jax.ShapeDtypeStruct((n, 1), F32),
    )(a, b)


def kernel(params, srna_node_id, mrna_node_id, edge_index_sm,
           edge_index_rev_sm, edge_index_mm, edge_index_rev_mm,
           edge_label_index):
    del srna_node_id, mrna_node_id
    ns = params['srna_emb'].shape[0]
    nm = params['mrna_emb'].shape[0]
    e = edge_index_sm.shape[1]
    el = edge_label_index.shape[1]

    NSP = _ru(ns, NC * NS * L)
    NMP = _ru(nm, NC * 2 * NS * L)
    EP = _ru(e, NS * 32 * 32)
    ELP = _ru(el, NC * NS * 64 * 16)

    xs = jnp.pad(params['srna_emb'].astype(F32), ((0, NSP - ns), (0, 0)))
    xm = jnp.pad(params['mrna_emb'].astype(F32), ((0, NMP - nm), (0, 0)))

    def eprep(ei, pad_dst):
        s = jnp.pad(ei[0].astype(I32), (0, EP - e)).reshape(EP // 32, 32)
        d = jnp.pad(ei[1].astype(I32), (0, EP - e),
                    constant_values=pad_dst).reshape(EP // 32, 32)
        return s, d

    s_sm, d_sm = eprep(edge_index_sm, nm)
    s_rsm, d_rsm = eprep(edge_index_rev_sm, ns)
    s_mm, d_mm = eprep(edge_index_mm, nm)
    s_rmm, d_rmm = eprep(edge_index_rev_mm, nm)

    counts = _make_counts(EP, (NMP, NMP, NSP, NMP))(d_sm, d_mm, d_rsm, d_rmm)
    c_sm = counts[0].reshape(NMP, 1)
    c_mm = counts[1].reshape(NMP, 1)
    c_rsm = counts[2].reshape(NSP, 1)
    c_rmm = counts[3].reshape(NMP, 1)

    seg = _make_segsum(EP, [
        (NSP, 1),
        (NMP, 2),
        (NMP, 2),
        (NMP, 2),
    ])

    for lyr in params['layers']:
        wl_sm, wr_sm, b_sm = lyr['sage_sm']
        wl_ms, wr_ms, b_ms = lyr['sage_ms']
        w_mm, b_mm = lyr['gcn_mm']
        w_rmm, b_rmm = lyr['gcn_rev_mm']

        h1, h2 = _gcn_h(xm, w_mm, w_rmm, c_mm, c_rmm)
        acc_s, acc_m1, acc_m2, acc_m3 = seg(
            xm, xs, h1, h2,
            s_rsm, s_sm, s_mm, s_rmm,
            d_rsm, d_sm, d_mm, d_rmm)
        xs = _sage_s(acc_s, c_rsm, xs, wl_ms, wr_ms, b_ms.reshape(1, D))
        xm = _m_update(acc_m1, c_sm, xm, wl_sm, wr_sm,
                       jnp.stack([b_sm, b_mm, b_rmm]),
                       acc_m2, h1, c_mm, acc_m3, h2, c_rmm)

    li0 = jnp.pad(edge_label_index[0].astype(I32),
                  (0, ELP - el)).reshape(ELP // 64, 64)
    li1 = jnp.pad(edge_label_index[1].astype(I32),
                  (0, ELP - el)).reshape(ELP // 64, 64)
    ef_s, ef_m = _make_pair_gather(ELP)(xs, xm, li0, li1)
    return _pair_dot(ef_s, ef_m)[:el, 0]


# speedup vs baseline: 1.6502x; 1.0086x over previous
import functools

import jax
import jax.numpy as jnp
from jax import lax
from jax.experimental import pallas as pl
from jax.experimental.pallas import tpu as pltpu
from jax.experimental.pallas import tpu_sc as plsc

D = 128
NC, NS, L = 2, 16, 16
F32 = jnp.float32
I32 = jnp.int32


@functools.cache
def _mesh():
    return plsc.VectorSubcoreMesh(core_axis_name="c", subcore_axis_name="s",
                                  num_cores=NC, num_subcores=NS)


def _ru(x, m):
    return (x + m - 1) // m * m


def _static_spans(total, step):
    out = []
    off = 0
    while off < total:
        w = min(step, total - off)
        out.append((off, w))
        off += w
    return out


def _make_counts(ep, sizes):
    rpe = ep // 32 // NS
    amax = max(sizes)

    def body(d0, d1, d2, d3, o0, o1, o2, o3, acc, dv, ones, zb, cb, sem):
        del sem
        cid = lax.axis_index("c")
        sid = lax.axis_index("s")
        def init16(i, c):
            zb[0, pl.ds(i * L, L)] = jnp.zeros((L,), F32)
            ones[0, pl.ds(lax.rem(i, jnp.int32(2)) * L, L)] = (
                jnp.ones((L,), F32))
            return c

        lax.fori_loop(0, 1024 // L, init16, 0)

        def job(dst_hbm, out_hbm, n):
            span = n // NS
            base = sid * span
            for (off, w) in _static_spans(span, 1024):
                pltpu.sync_copy(zb.at[0, pl.ds(0, w)],
                                acc.at[pl.ds(base + off, w)])
            plsc.subcore_barrier()
            pltpu.sync_copy(dst_hbm.at[pl.ds(sid * rpe, rpe)], dv)

            def it(j, c):
                pltpu.sync_copy(ones.at[0], acc.at[dv.at[j]], add=True)
                return c

            lax.fori_loop(0, rpe, it, 0)
            plsc.subcore_barrier()
            pltpu.sync_copy(acc.at[pl.ds(base, span)], cb.at[pl.ds(0, span)])
            pltpu.sync_copy(cb.at[pl.ds(0, span)],
                            out_hbm.at[pl.ds(base, span)])
            plsc.subcore_barrier()

        @pl.when(cid == 0)
        def _():
            job(d0, o0, sizes[0])
            job(d1, o1, sizes[1])

        @pl.when(cid == 1)
        def _():
            job(d2, o2, sizes[2])
            job(d3, o3, sizes[3])

    return pl.kernel(
        body,
        out_type=[jax.ShapeDtypeStruct((s,), F32) for s in sizes],
        mesh=_mesh(),
        scratch_types=[
            pltpu.VMEM_SHARED((amax,), F32),
            pltpu.VMEM((rpe, 32), I32),
            pltpu.VMEM((1, 32), F32),
            pltpu.VMEM((1, 1024), F32),
            pltpu.VMEM((amax // NS,), F32),
            pltpu.SemaphoreType.DMA,
        ],
    )


def _make_segsum(ep, jobs):
    gw = 32
    eb = 16
    ept = ep // NS
    nblk = ept // 32 // eb
    amax = max(nd_p // (NC * npc) for (nd_p, npc) in jobs) + 8
    fcap = _ru(ept + 4 * gw + 8, 8)
    trash_slot = fcap - 8
    assert ept % (32 * eb) == 0

    def body(*refs):
        nj = len(jobs)
        xs = refs[0:nj]
        srcs = refs[nj:2 * nj]
        dsts = refs[2 * nj:3 * nj]
        outs = refs[3 * nj:4 * nj]
        (acc, sv, dv, fbuf, r0, r1, r2, x0, x1, x2, u0, u1, u2, zb,
         g0, g1, g2, s0, s1, s2, zs) = refs[4 * nj:]
        rows = (r0, r1, r2)
        sidx = (x0, x1, x2)
        usrc = (u0, u1, u2)
        gs = (g0, g1, g2)
        ss = (s0, s1, s2)
        cid = lax.axis_index("c")
        sid = lax.axis_index("s")

        def zinit(i, c):
            zb[i // (D // L), pl.ds(lax.rem(i, jnp.int32(D // L)) * L, L)] = (
                jnp.zeros((L,), F32))
            return c

        lax.fori_loop(0, 4 * (D // L), zinit, 0)
        cp = pltpu.async_copy

        for j, (nd_p, npc) in enumerate(jobs):
            x_hbm, src_hbm, dst_hbm, out_hbm = xs[j], srcs[j], dsts[j], outs[j]
            chunk = nd_p // (NC * npc)
            rpt = chunk // NS
            nz = rpt // 4
            for cj in range(npc):
                lo = (cid * npc + cj) * chunk
                def ziss(i, c):
                    cp(zb, acc.at[pl.ds(sid * rpt + i * 4, 4)], zs)
                    return c

                def zdrn(i, c):
                    pltpu.make_async_copy(
                        zb, acc.at[pl.ds(sid * rpt, 4)], zs).wait()
                    return c

                lax.fori_loop(0, nz, ziss, 0)
                lax.fori_loop(0, nz, zdrn, 0)
                plsc.subcore_barrier()

                iota = lax.iota(I32, L)

                def blkfilt(blk, ptr):
                    rr = (sid * nblk + blk) * eb
                    pltpu.sync_copy(src_hbm.at[pl.ds(rr, eb)], sv)
                    pltpu.sync_copy(dst_hbm.at[pl.ds(rr, eb)], dv)

                    def filt(r, ptr2):
                        res = []
                        for c in (0, L):
                            dd = dv[r, pl.ds(c, L)]
                            ss_ = sv[r, pl.ds(c, L)]
                            m = (dd >= lo) & (dd < lo + chunk)
                            mi = m.astype(I32)
                            p = mi
                            for sh in (1, 2, 4, 8):
                                g = p[jnp.maximum(iota - sh, 0)]
                                p = p + jnp.where(iota >= sh, g, 0)
                            v = ss_ | ((dd - lo) << 16)
                            res.append((m, p - mi, p[L - 1], v))
                        m0, e0, c0, v0 = res[0]
                        m1, e1, c1, v1 = res[1]
                        pos0 = jnp.where(m0, ptr2 + e0, jnp.int32(trash_slot))
                        plsc.store_scatter(fbuf, [pos0], v0)
                        pos1 = jnp.where(m1, ptr2 + c0 + e1,
                                         jnp.int32(trash_slot))
                        plsc.store_scatter(fbuf, [pos1], v1)
                        return ptr2 + c0 + c1

                    return lax.fori_loop(0, eb, filt, ptr)

                n = lax.fori_loop(0, nblk, blkfilt, jnp.int32(0))
                vpad = jnp.full((L,), chunk << 16, I32)
                for t in range(4 * gw // L):
                    fbuf[pl.ds(n + t * L, L)] = vpad
                n3 = jnp.maximum(lax.div(n + 3 * gw - 1, jnp.int32(3 * gw)),
                                 jnp.int32(1))

                def unpack(u, b):
                    for k in range(gw // L):
                        vv = fbuf[pl.ds(u * gw + k * L, L)]
                        usrc[b][pl.ds(k * L, L)] = vv & jnp.int32(0xFFFF)
                        sidx[b][pl.ds(k * L, L)] = (
                            lax.shift_right_logical(vv, 16))

                def unit(u, b, wait_prev, issue_next):
                    pltpu.make_async_copy(x_hbm.at[usrc[b]], rows[b],
                                          gs[b]).wait()
                    cp(rows[b], acc.at[sidx[b]], ss[b], add=True)
                    bp = (b + 2) % 3
                    if wait_prev is not None:
                        def _w():
                            pltpu.make_async_copy(rows[bp], acc.at[sidx[bp]],
                                                  ss[bp]).wait()
                        if wait_prev is True:
                            _w()
                        else:
                            pl.when(wait_prev)(_w)
                    if issue_next is not None:
                        def _i():
                            unpack(u + 2, bp)
                            cp(x_hbm.at[usrc[bp]], rows[bp], gs[bp])
                        if issue_next is True:
                            _i()
                        else:
                            pl.when(issue_next)(_i)

                unpack(0, 0)
                cp(x_hbm.at[usrc[0]], rows[0], gs[0])
                unpack(1, 1)
                cp(x_hbm.at[usrc[1]], rows[1], gs[1])

                def step(i, c2):
                    u = 3 * i
                    unit(u, 0, i > 0, True)
                    unit(u + 1, 1, True, i < n3 - 1)
                    unit(u + 2, 2, True, i < n3 - 1)
                    return c2

                lax.fori_loop(0, n3, step, 0)
                pltpu.make_async_copy(rows[2], acc.at[sidx[2]], ss[2]).wait()
                plsc.subcore_barrier()
                pltpu.sync_copy(acc.at[pl.ds(sid * rpt, rpt)],
                                out_hbm.at[pl.ds(lo + sid * rpt, rpt)])
                plsc.subcore_barrier()

    return pl.kernel(
        body,
        out_type=[jax.ShapeDtypeStruct((nd_p, D), F32)
                  for (nd_p, _) in jobs],
        mesh=_mesh(),
        scratch_types=(
            [pltpu.VMEM_SHARED((amax, D), F32),
             pltpu.VMEM((eb, 32), I32),
             pltpu.VMEM((eb, 32), I32),
             pltpu.VMEM((fcap,), I32)]
            + [pltpu.VMEM((gw, D), F32)] * 3
            + [pltpu.VMEM((gw,), I32)] * 3
            + [pltpu.VMEM((gw,), I32)] * 3
            + [pltpu.VMEM((4, D), F32)]
            + [pltpu.SemaphoreType.DMA] * 7
        ),
        compiler_params=pltpu.CompilerParams(needs_layout_passes=False),
    )


def _make_pair_gather(elp):
    per_w = elp // (NC * NS)
    n_g = per_w // 64

    def body(xs_hbm, xm_hbm, li0_hbm, li1_hbm, es_hbm, em_hbm,
             iv, rowsa, rowsb, sema, semb):
        cid = lax.axis_index("c")
        sid = lax.axis_index("s")
        w = cid * NS + sid
        r0 = w * n_g
        rows = (rowsa, rowsb)
        sems = (sema, semb)

        for (src, idx_hbm, out_hbm) in ((xs_hbm, li0_hbm, es_hbm),
                                        (xm_hbm, li1_hbm, em_hbm)):
            pltpu.sync_copy(idx_hbm.at[pl.ds(r0, n_g)], iv)
            pltpu.async_copy(src.at[iv.at[0]], rows[0], sems[0])

            def it(i, c):
                g = 2 * i
                pltpu.make_async_copy(src.at[iv.at[0]], rows[0],
                                      sems[0]).wait()
                pltpu.async_copy(src.at[iv.at[g + 1]], rows[1], sems[1])
                pltpu.sync_copy(rows[0],
                                out_hbm.at[pl.ds((r0 + g) * 64, 64)])
                pltpu.make_async_copy(src.at[iv.at[0]], rows[1],
                                      sems[1]).wait()

                @pl.when(i < n_g // 2 - 1)
                def _():
                    pltpu.async_copy(src.at[iv.at[g + 2]], rows[0], sems[0])

                pltpu.sync_copy(rows[1],
                                out_hbm.at[pl.ds((r0 + g + 1) * 64, 64)])
                return c

            lax.fori_loop(0, n_g // 2, it, 0)

    return pl.kernel(
        body,
        out_type=[jax.ShapeDtypeStruct((elp, D), F32),
                  jax.ShapeDtypeStruct((elp, D), F32)],
        mesh=_mesh(),
        scratch_types=[
            pltpu.VMEM((n_g, 64), I32),
            pltpu.VMEM((64, D), F32),
            pltpu.VMEM((64, D), F32),
            pltpu.SemaphoreType.DMA,
            pltpu.SemaphoreType.DMA,
        ],
    )


_BLK = 1024


def _gcn_h_body(xm, w1, w2, g1, g2, h1, h2):
    x = xm[...]
    d1 = lax.rsqrt(g1[...] + 1.0)
    d2 = lax.rsqrt(g2[...] + 1.0)
    h1[...] = d1 * jnp.dot(x, w1[...], preferred_element_type=F32)
    h2[...] = d2 * jnp.dot(x, w2[...], preferred_element_type=F32)


def _gcn_h(xm, w1, w2, g1, g2):
    n = xm.shape[0]
    bs_row = pl.BlockSpec((_BLK, D), lambda i: (i, 0))
    bs_w = pl.BlockSpec((D, D), lambda i: (0, 0))
    bs_g = pl.BlockSpec((_BLK, 1), lambda i: (i, 0))
    return pl.pallas_call(
        _gcn_h_body,
        grid=(n // _BLK,),
        in_specs=[bs_row, bs_w, bs_w, bs_g, bs_g],
        out_specs=[bs_row, bs_row],
        out_shape=[jax.ShapeDtypeStruct((n, D), F32)] * 2,
    )(xm, w1, w2, g1, g2)


def _sage_s_body(a, cnt, x, wl, wr, b, o):
    agg = a[...] / jnp.maximum(cnt[...], 1.0)
    o[...] = jax.nn.relu(jnp.dot(agg, wl[...], preferred_element_type=F32)
                         + jnp.dot(x[...], wr[...], preferred_element_type=F32)
                         + b[...])


def _sage_s(acc, cnt, x, wl, wr, b):
    n = x.shape[0]
    bs_row = pl.BlockSpec((_BLK, D), lambda i: (i, 0))
    bs_w = pl.BlockSpec((D, D), lambda i: (0, 0))
    bs_g = pl.BlockSpec((_BLK, 1), lambda i: (i, 0))
    bs_b = pl.BlockSpec((1, D), lambda i: (0, 0))
    return pl.pallas_call(
        _sage_s_body,
        grid=(n // _BLK,),
        in_specs=[bs_row, bs_g, bs_row, bs_w, bs_w, bs_b],
        out_specs=bs_row,
        out_shape=jax.ShapeDtypeStruct((n, D), F32),
    )(acc, cnt, x, wl, wr, b)


def _m_update_body(a1, cnt, x, wl, wr, bb, a2, h1, g1, a3, h2, g2, o):
    agg = a1[...] / jnp.maximum(cnt[...], 1.0)
    t = (jnp.dot(agg, wl[...], preferred_element_type=F32)
         + jnp.dot(x[...], wr[...], preferred_element_type=F32)
         + bb[0:1, :] + bb[1:2, :] + bb[2:3, :])
    d1 = lax.rsqrt(g1[...] + 1.0)
    d2 = lax.rsqrt(g2[...] + 1.0)
    t = t + d1 * (a2[...] + h1[...]) + d2 * (a3[...] + h2[...])
    o[...] = jax.nn.relu(t)


def _m_update(a1, cnt, x, wl, wr, bb, a2, h1, g1, a3, h2, g2):
    n = x.shape[0]
    bs_row = pl.BlockSpec((_BLK, D), lambda i: (i, 0))
    bs_w = pl.BlockSpec((D, D), lambda i: (0, 0))
    bs_g = pl.BlockSpec((_BLK, 1), lambda i: (i, 0))
    bs_b = pl.BlockSpec((3, D), lambda i: (0, 0))
    return pl.pallas_call(
        _m_update_body,
        grid=(n // _BLK,),
        in_specs=[bs_row, bs_g, bs_row, bs_w, bs_w, bs_b,
                  bs_row, bs_row, bs_g, bs_row, bs_row, bs_g],
        out_specs=bs_row,
        out_shape=jax.ShapeDtypeStruct((n, D), F32),
    )(a1, cnt, x, wl, wr, bb, a2, h1, g1, a3, h2, g2)


def _dot_body(a, b, o):
    o[...] = jnp.sum(a[...] * b[...], axis=1, keepdims=True)


def _pair_dot(a, b):
    n = a.shape[0]
    blk = 2048
    bs_row = pl.BlockSpec((blk, D), lambda i: (i, 0))
    bs_o = pl.BlockSpec((blk, 1), lambda i: (i, 0))
    return pl.pallas_call(
        _dot_body,
        grid=(n // blk,),
        in_specs=[bs_row, bs_row],
        out_specs=bs_o,
        out_shape=jax.ShapeDtypeStruct((n, 1), F32),
    )(a, b)


def kernel(params, srna_node_id, mrna_node_id, edge_index_sm,
           edge_index_rev_sm, edge_index_mm, edge_index_rev_mm,
           edge_label_index):
    del srna_node_id, mrna_node_id
    ns = params['srna_emb'].shape[0]
    nm = params['mrna_emb'].shape[0]
    e = edge_index_sm.shape[1]
    el = edge_label_index.shape[1]

    NSP = _ru(ns, NC * NS * L)
    NMP = _ru(nm, NC * 2 * NS * L)
    EP = _ru(e, NS * 32 * 32)
    ELP = _ru(el, NC * NS * 64 * 16)

    xs = jnp.pad(params['srna_emb'].astype(F32), ((0, NSP - ns), (0, 0)))
    xm = jnp.pad(params['mrna_emb'].astype(F32), ((0, NMP - nm), (0, 0)))

    def eprep(ei, pad_dst):
        s = jnp.pad(ei[0].astype(I32), (0, EP - e)).reshape(EP // 32, 32)
        d = jnp.pad(ei[1].astype(I32), (0, EP - e),
                    constant_values=pad_dst).reshape(EP // 32, 32)
        return s, d

    s_sm, d_sm = eprep(edge_index_sm, nm)
    s_rsm, d_rsm = eprep(edge_index_rev_sm, ns)
    s_mm, d_mm = eprep(edge_index_mm, nm)
    s_rmm, d_rmm = eprep(edge_index_rev_mm, nm)

    counts = _make_counts(EP, (NMP, NMP, NSP, NMP))(d_sm, d_mm, d_rsm, d_rmm)
    c_sm = counts[0].reshape(NMP, 1)
    c_mm = counts[1].reshape(NMP, 1)
    c_rsm = counts[2].reshape(NSP, 1)
    c_rmm = counts[3].reshape(NMP, 1)

    seg_a = _make_segsum(EP, [
        (NSP, 1),
        (NMP, 2),
    ])
    seg_b = _make_segsum(EP, [
        (NMP, 2),
        (NMP, 2),
    ])

    for lyr in params['layers']:
        wl_sm, wr_sm, b_sm = lyr['sage_sm']
        wl_ms, wr_ms, b_ms = lyr['sage_ms']
        w_mm, b_mm = lyr['gcn_mm']
        w_rmm, b_rmm = lyr['gcn_rev_mm']

        h1, h2 = _gcn_h(xm, w_mm, w_rmm, c_mm, c_rmm)
        acc_s, acc_m1 = seg_a(xm, xs, s_rsm, s_sm, d_rsm, d_sm)
        acc_m2, acc_m3 = seg_b(h1, h2, s_mm, s_rmm, d_mm, d_rmm)
        xs = _sage_s(acc_s, c_rsm, xs, wl_ms, wr_ms, b_ms.reshape(1, D))
        xm = _m_update(acc_m1, c_sm, xm, wl_sm, wr_sm,
                       jnp.stack([b_sm, b_mm, b_rmm]),
                       acc_m2, h1, c_mm, acc_m3, h2, c_rmm)

    li0 = jnp.pad(edge_label_index[0].astype(I32),
                  (0, ELP - el)).reshape(ELP // 64, 64)
    li1 = jnp.pad(edge_label_index[1].astype(I32),
                  (0, ELP - el)).reshape(ELP // 64, 64)
    ef_s, ef_m = _make_pair_gather(ELP)(xs, xm, li0, li1)
    return _pair_dot(ef_s, ef_m)[:el, 0]
